# Initial kernel scaffold; baseline (speedup 1.0000x reference)
import jax
import jax.numpy as jnp
from jax.experimental import pallas as pl


def kernel(x, edge_index, W1_self, W1_neigh, b1, W2_self, W2_neigh, b2):
    raise NotImplementedError("write your pallas kernel here")


# speedup vs baseline: 10.1163x; 10.1163x over previous
import functools

import jax
import jax.numpy as jnp
from jax import lax
from jax.experimental import pallas as pl
from jax.experimental.pallas import tpu as pltpu
from jax.experimental.pallas import tpu_sc as plsc

_N = 10000
_D = 128
_H = 16
_C = 40
_E = 320000
_W1 = 32

_NC = 2
_NS = 16
_NW = _NC * _NS
_M = 128
_R = 80
_EP = _NW * _R * _M
_NPAD = 10112
_STRIPE = _NPAD // _NS
_DUMMY = 10008

_BLK = 2000


def _tc_pre_body(x_ref, ws_ref, wn_ref, b1_ref, c1_ref, xs_ref, xn_ref):
    x = x_ref[...]
    xs_ref[...] = jnp.dot(x, ws_ref[...], preferred_element_type=jnp.float32) + b1_ref[...]
    xn_ref[...] = jnp.dot(x, wn_ref[...], preferred_element_type=jnp.float32) + c1_ref[...]


def _tc_pre(x, wsT, wnT, b1r, c1r):
    return pl.pallas_call(
        _tc_pre_body,
        grid=(_N // _BLK,),
        in_specs=[
            pl.BlockSpec((_BLK, _D), lambda i: (i, 0)),
            pl.BlockSpec((_D, _H), lambda i: (0, 0)),
            pl.BlockSpec((_D, _W1), lambda i: (0, 0)),
            pl.BlockSpec((1, _H), lambda i: (0, 0)),
            pl.BlockSpec((1, _W1), lambda i: (0, 0)),
        ],
        out_specs=[
            pl.BlockSpec((_BLK, _H), lambda i: (i, 0)),
            pl.BlockSpec((_BLK, _W1), lambda i: (i, 0)),
        ],
        out_shape=[
            jax.ShapeDtypeStruct((_N, _H), jnp.float32),
            jax.ShapeDtypeStruct((_N, _W1), jnp.float32),
        ],
    )(x, wsT, wnT, b1r, c1r)


def _sc_agg(table, src_rows, dst_rows, zeros, width):
    @functools.partial(
        pl.kernel,
        out_type=jax.ShapeDtypeStruct((_NC, _NPAD, width), jnp.float32),
        mesh=plsc.VectorSubcoreMesh(core_axis_name="c", subcore_axis_name="s"),
        compiler_params=pltpu.CompilerParams(use_tc_tiling_on_sc=False),
        scratch_types=[
            pltpu.VMEM((_R, _M), jnp.int32),
            pltpu.VMEM((_R, _M), jnp.int32),
            pltpu.VMEM((_M, width), jnp.float32),
            pltpu.VMEM((_STRIPE, width), jnp.float32),
            pltpu.VMEM_SHARED((_NPAD, width), jnp.float32),
            pltpu.SemaphoreType.DMA,
        ],
    )
    def agg(table_hbm, src_hbm, dst_hbm, z_hbm, out_hbm,
            src_v, dst_v, rows_v, strip_v, acc_sh, sem):
        c = lax.axis_index("c")
        s = lax.axis_index("s")
        pltpu.sync_copy(z_hbm.at[pl.ds(s * _STRIPE, _STRIPE)], strip_v)
        pltpu.sync_copy(strip_v, acc_sh.at[pl.ds(s * _STRIPE, _STRIPE)])
        plsc.subcore_barrier()
        wid = s * _NC + c
        pltpu.sync_copy(src_hbm.at[pl.ds(wid * _R, _R)], src_v)
        pltpu.sync_copy(dst_hbm.at[pl.ds(wid * _R, _R)], dst_v)

        def chunk(j, carry):
            pltpu.async_copy(table_hbm.at[src_v.at[j]], rows_v, sem).wait()
            pltpu.sync_copy(rows_v, acc_sh.at[dst_v.at[j]], add=True)
            return carry

        lax.fori_loop(0, _R, chunk, 0)
        plsc.subcore_barrier()
        pltpu.sync_copy(acc_sh.at[pl.ds(s * _STRIPE, _STRIPE)], strip_v)
        pltpu.sync_copy(strip_v, out_hbm.at[c, pl.ds(s * _STRIPE, _STRIPE)])

    return agg(table, src_rows, dst_rows, zeros)


def _tc_mid_body(xs_ref, p_ref, w2s_ref, b2_ref, h_ref, hs_ref):
    p = p_ref[0] + p_ref[1]
    den = jnp.maximum(p[:, _H:_H + 1], 1.0)
    h = jnp.maximum(xs_ref[...] + p[:, :_H] / den, 0.0)
    h_ref[...] = h
    hs_ref[...] = jnp.dot(h, w2s_ref[...], preferred_element_type=jnp.float32) + b2_ref[...]


def _tc_mid(xs, p32, w2sT, b2r):
    return pl.pallas_call(
        _tc_mid_body,
        grid=(_N // _BLK,),
        in_specs=[
            pl.BlockSpec((_BLK, _H), lambda i: (i, 0)),
            pl.BlockSpec((_NC, _BLK, _W1), lambda i: (0, i, 0)),
            pl.BlockSpec((_H, _C), lambda i: (0, 0)),
            pl.BlockSpec((1, _C), lambda i: (0, 0)),
        ],
        out_specs=[
            pl.BlockSpec((_BLK, _H), lambda i: (i, 0)),
            pl.BlockSpec((_BLK, _C), lambda i: (i, 0)),
        ],
        out_shape=[
            jax.ShapeDtypeStruct((_N, _H), jnp.float32),
            jax.ShapeDtypeStruct((_N, _C), jnp.float32),
        ],
    )(xs, p32, w2sT, b2r)


def _tc_out_body(hs_ref, q_ref, p_ref, w2n_ref, o_ref):
    p = p_ref[0] + p_ref[1]
    den = jnp.maximum(p[:, _H:_H + 1], 1.0)
    agg = q_ref[0] + q_ref[1]
    o_ref[...] = hs_ref[...] + jnp.dot(agg, w2n_ref[...], preferred_element_type=jnp.float32) / den


def _tc_out(hs, q16, p32, w2nT):
    return pl.pallas_call(
        _tc_out_body,
        grid=(_N // _BLK,),
        in_specs=[
            pl.BlockSpec((_BLK, _C), lambda i: (i, 0)),
            pl.BlockSpec((_NC, _BLK, _H), lambda i: (0, i, 0)),
            pl.BlockSpec((_NC, _BLK, _W1), lambda i: (0, i, 0)),
            pl.BlockSpec((_H, _C), lambda i: (0, 0)),
        ],
        out_specs=pl.BlockSpec((_BLK, _C), lambda i: (i, 0)),
        out_shape=jax.ShapeDtypeStruct((_N, _C), jnp.float32),
    )(hs, q16, p32, w2nT)


def kernel(x, edge_index, W1_self, W1_neigh, b1, W2_self, W2_neigh, b2):
    src = edge_index[0].astype(jnp.int32)
    dst = edge_index[1].astype(jnp.int32)
    pad = _EP - _E
    src_rows = jnp.concatenate([src, jnp.zeros((pad,), jnp.int32)]).reshape(_NW * _R, _M)
    dst_rows = jnp.concatenate([dst, jnp.full((pad,), _DUMMY, jnp.int32)]).reshape(_NW * _R, _M)
    wsT = W1_self.T
    wnT = jnp.zeros((_D, _W1), jnp.float32).at[:, :_H].set(W1_neigh.T)
    c1 = jnp.zeros((1, _W1), jnp.float32).at[0, _H].set(1.0)
    z32 = jnp.zeros((_NPAD, _W1), jnp.float32)
    z16 = jnp.zeros((_NPAD, _H), jnp.float32)

    xs, xn = _tc_pre(x, wsT, wnT, b1.reshape(1, _H), c1)
    p32 = _sc_agg(xn, src_rows, dst_rows, z32, _W1)
    h, hs = _tc_mid(xs, p32, W2_self.T, b2.reshape(1, _C))
    q16 = _sc_agg(h, src_rows, dst_rows, z16, _H)
    return _tc_out(hs, q16, p32, W2_neigh.T)


# speedup vs baseline: 11.9473x; 1.1810x over previous
import functools

import jax
import jax.numpy as jnp
from jax import lax
from jax.experimental import pallas as pl
from jax.experimental.pallas import tpu as pltpu
from jax.experimental.pallas import tpu_sc as plsc

_N = 10000
_D = 128
_H = 16
_C = 40
_E = 320000
_W1 = 32

_NC = 2
_NS = 16
_NW = _NC * _NS
_M = 128
_R = 80
_EP = _NW * _R * _M
_NPAD = 10112
_STRIPE = _NPAD // _NS
_DUMMY = 10008
_K = 4

_BLK = 2000


def _tc_pre_body(x_ref, ws_ref, wn_ref, b1_ref, c1_ref, xs_ref, xn_ref):
    x = x_ref[...]
    xs_ref[...] = jnp.dot(x, ws_ref[...], preferred_element_type=jnp.float32) + b1_ref[...]
    xn_ref[...] = jnp.dot(x, wn_ref[...], preferred_element_type=jnp.float32) + c1_ref[...]


def _tc_pre(x, wsT, wnT, b1r, c1r):
    return pl.pallas_call(
        _tc_pre_body,
        grid=(_N // _BLK,),
        in_specs=[
            pl.BlockSpec((_BLK, _D), lambda i: (i, 0)),
            pl.BlockSpec((_D, _H), lambda i: (0, 0)),
            pl.BlockSpec((_D, _W1), lambda i: (0, 0)),
            pl.BlockSpec((1, _H), lambda i: (0, 0)),
            pl.BlockSpec((1, _W1), lambda i: (0, 0)),
        ],
        out_specs=[
            pl.BlockSpec((_BLK, _H), lambda i: (i, 0)),
            pl.BlockSpec((_BLK, _W1), lambda i: (i, 0)),
        ],
        out_shape=[
            jax.ShapeDtypeStruct((_N, _H), jnp.float32),
            jax.ShapeDtypeStruct((_N, _W1), jnp.float32),
        ],
    )(x, wsT, wnT, b1r, c1r)


def _sc_agg(table, src_rows, dst_rows, zeros, width):
    @functools.partial(
        pl.kernel,
        out_type=jax.ShapeDtypeStruct((_NC, _NPAD, width), jnp.float32),
        mesh=plsc.VectorSubcoreMesh(core_axis_name="c", subcore_axis_name="s"),
        compiler_params=pltpu.CompilerParams(use_tc_tiling_on_sc=False),
        scratch_types=[
            pltpu.VMEM((_R, _M), jnp.int32),
            pltpu.VMEM((_R, _M), jnp.int32),
            pltpu.VMEM((_K, _M, width), jnp.float32),
            pltpu.VMEM((_STRIPE, width), jnp.float32),
            pltpu.VMEM_SHARED((_NPAD, width), jnp.float32),
            [pltpu.SemaphoreType.DMA] * _K,
            [pltpu.SemaphoreType.DMA] * _K,
        ],
    )
    def agg(table_hbm, src_hbm, dst_hbm, z_hbm, out_hbm,
            src_v, dst_v, rows_v, strip_v, acc_sh, gsems, ssems):
        c = lax.axis_index("c")
        s = lax.axis_index("s")
        pltpu.sync_copy(z_hbm.at[pl.ds(s * _STRIPE, _STRIPE)], strip_v)
        pltpu.sync_copy(strip_v, acc_sh.at[pl.ds(s * _STRIPE, _STRIPE)])
        plsc.subcore_barrier()
        wid = s * _NC + c
        pltpu.sync_copy(src_hbm.at[pl.ds(wid * _R, _R)], src_v)
        pltpu.sync_copy(dst_hbm.at[pl.ds(wid * _R, _R)], dst_v)

        def pipe(g, carry):
            gd = []
            for b in range(_K):
                j = g * _K + b
                gd.append(pltpu.async_copy(
                    table_hbm.at[src_v.at[j]], rows_v.at[b], gsems[b]))
            sd = []
            for b in range(_K):
                j = g * _K + b
                gd[b].wait()
                sd.append(pltpu.async_copy(
                    rows_v.at[b], acc_sh.at[dst_v.at[j]], ssems[b], add=True))
            for b in range(_K):
                sd[b].wait()
            return carry

        lax.fori_loop(0, _R // _K, pipe, 0)
        plsc.subcore_barrier()
        pltpu.sync_copy(acc_sh.at[pl.ds(s * _STRIPE, _STRIPE)], strip_v)
        pltpu.sync_copy(strip_v, out_hbm.at[c, pl.ds(s * _STRIPE, _STRIPE)])

    return agg(table, src_rows, dst_rows, zeros)


def _tc_mid_body(xs_ref, p_ref, w2s_ref, b2_ref, h_ref, hs_ref):
    p = p_ref[0] + p_ref[1]
    den = jnp.maximum(p[:, _H:_H + 1], 1.0)
    h = jnp.maximum(xs_ref[...] + p[:, :_H] / den, 0.0)
    h_ref[...] = h
    hs_ref[...] = jnp.dot(h, w2s_ref[...], preferred_element_type=jnp.float32) + b2_ref[...]


def _tc_mid(xs, p32, w2sT, b2r):
    return pl.pallas_call(
        _tc_mid_body,
        grid=(_N // _BLK,),
        in_specs=[
            pl.BlockSpec((_BLK, _H), lambda i: (i, 0)),
            pl.BlockSpec((_NC, _BLK, _W1), lambda i: (0, i, 0)),
            pl.BlockSpec((_H, _C), lambda i: (0, 0)),
            pl.BlockSpec((1, _C), lambda i: (0, 0)),
        ],
        out_specs=[
            pl.BlockSpec((_BLK, _H), lambda i: (i, 0)),
            pl.BlockSpec((_BLK, _C), lambda i: (i, 0)),
        ],
        out_shape=[
            jax.ShapeDtypeStruct((_N, _H), jnp.float32),
            jax.ShapeDtypeStruct((_N, _C), jnp.float32),
        ],
    )(xs, p32, w2sT, b2r)


def _tc_out_body(hs_ref, q_ref, p_ref, w2n_ref, o_ref):
    p = p_ref[0] + p_ref[1]
    den = jnp.maximum(p[:, _H:_H + 1], 1.0)
    agg = q_ref[0] + q_ref[1]
    o_ref[...] = hs_ref[...] + jnp.dot(agg, w2n_ref[...], preferred_element_type=jnp.float32) / den


def _tc_out(hs, q16, p32, w2nT):
    return pl.pallas_call(
        _tc_out_body,
        grid=(_N // _BLK,),
        in_specs=[
            pl.BlockSpec((_BLK, _C), lambda i: (i, 0)),
            pl.BlockSpec((_NC, _BLK, _H), lambda i: (0, i, 0)),
            pl.BlockSpec((_NC, _BLK, _W1), lambda i: (0, i, 0)),
            pl.BlockSpec((_H, _C), lambda i: (0, 0)),
        ],
        out_specs=pl.BlockSpec((_BLK, _C), lambda i: (i, 0)),
        out_shape=jax.ShapeDtypeStruct((_N, _C), jnp.float32),
    )(hs, q16, p32, w2nT)


def kernel(x, edge_index, W1_self, W1_neigh, b1, W2_self, W2_neigh, b2):
    src = edge_index[0].astype(jnp.int32)
    dst = edge_index[1].astype(jnp.int32)
    pad = _EP - _E
    src_rows = jnp.concatenate([src, jnp.zeros((pad,), jnp.int32)]).reshape(_NW * _R, _M)
    dst_rows = jnp.concatenate([dst, jnp.full((pad,), _DUMMY, jnp.int32)]).reshape(_NW * _R, _M)
    wsT = W1_self.T
    wnT = jnp.zeros((_D, _W1), jnp.float32).at[:, :_H].set(W1_neigh.T)
    c1 = jnp.zeros((1, _W1), jnp.float32).at[0, _H].set(1.0)
    z32 = jnp.zeros((_NPAD, _W1), jnp.float32)
    z16 = jnp.zeros((_NPAD, _H), jnp.float32)

    xs, xn = _tc_pre(x, wsT, wnT, b1.reshape(1, _H), c1)
    p32 = _sc_agg(xn, src_rows, dst_rows, z32, _W1)
    h, hs = _tc_mid(xs, p32, W2_self.T, b2.reshape(1, _C))
    q16 = _sc_agg(h, src_rows, dst_rows, z16, _H)
    return _tc_out(hs, q16, p32, W2_neigh.T)


# speedup vs baseline: 22.7533x; 1.9045x over previous
import functools

import jax
import jax.numpy as jnp
from jax import lax
from jax.experimental import pallas as pl
from jax.experimental.pallas import tpu as pltpu
from jax.experimental.pallas import tpu_sc as plsc

_N = 10000
_D = 128
_H = 16
_C = 40
_E = 320000
_W1 = 32

_NC = 2
_NS = 16
_NW = _NC * _NS
_M = 128
_R = 80
_EP = _NW * _R * _M
_NPAD = 10112
_STRIPE = _NPAD // _NS
_K = 8

_BLK = 2000


def _tc_pre_body(x_ref, ws_ref, wn_ref, b1_ref, c1_ref, xs_ref, xn_ref):
    x = x_ref[...]
    xs_ref[...] = jnp.dot(x, ws_ref[...], preferred_element_type=jnp.float32) + b1_ref[...]
    xn_ref[...] = jnp.dot(x, wn_ref[...], preferred_element_type=jnp.float32) + c1_ref[...]


def _tc_pre(x, wsT, wnT, b1r, c1r):
    return pl.pallas_call(
        _tc_pre_body,
        grid=(_N // _BLK,),
        in_specs=[
            pl.BlockSpec((_BLK, _D), lambda i: (i, 0)),
            pl.BlockSpec((_D, _H), lambda i: (0, 0)),
            pl.BlockSpec((_D, _W1), lambda i: (0, 0)),
            pl.BlockSpec((1, _H), lambda i: (0, 0)),
            pl.BlockSpec((1, _W1), lambda i: (0, 0)),
        ],
        out_specs=[
            pl.BlockSpec((_BLK, _H), lambda i: (i, 0)),
            pl.BlockSpec((_BLK, _W1), lambda i: (i, 0)),
        ],
        out_shape=[
            jax.ShapeDtypeStruct((_N, _H), jnp.float32),
            jax.ShapeDtypeStruct((_N, _W1), jnp.float32),
        ],
    )(x, wsT, wnT, b1r, c1r)


def _sc_agg(table, src_rows, dst_rows, zeros, width):
    @functools.partial(
        pl.kernel,
        out_type=jax.ShapeDtypeStruct((_NC, _NPAD, width), jnp.float32),
        mesh=plsc.VectorSubcoreMesh(core_axis_name="c", subcore_axis_name="s"),
        compiler_params=pltpu.CompilerParams(use_tc_tiling_on_sc=False),
        scratch_types=[
            pltpu.VMEM((_R, _M), jnp.int32),
            pltpu.VMEM((_R, _M), jnp.int32),
            pltpu.VMEM((_K, _M, width), jnp.float32),
            pltpu.VMEM((_STRIPE, width), jnp.float32),
            pltpu.VMEM_SHARED((_NPAD, width), jnp.float32),
            [pltpu.SemaphoreType.DMA] * _K,
            [pltpu.SemaphoreType.DMA] * _K,
        ],
    )
    def agg(table_hbm, src_hbm, dst_hbm, z_hbm, out_hbm,
            src_v, dst_v, rows_v, strip_v, acc_sh, gsems, ssems):
        c = lax.axis_index("c")
        s = lax.axis_index("s")
        pltpu.sync_copy(z_hbm.at[pl.ds(s * _STRIPE, _STRIPE)], strip_v)
        pltpu.sync_copy(strip_v, acc_sh.at[pl.ds(s * _STRIPE, _STRIPE)])
        plsc.subcore_barrier()
        wid = s * _NC + c
        pltpu.sync_copy(src_hbm.at[pl.ds(wid * _R, _R)], src_v)
        pltpu.sync_copy(dst_hbm.at[pl.ds(wid * _R, _R)], dst_v)

        def pipe(g, carry):
            gd = []
            for b in range(_K):
                j = g * _K + b
                gd.append(pltpu.async_copy(
                    table_hbm.at[src_v.at[j]], rows_v.at[b], gsems[b]))
            sd = []
            for b in range(_K):
                j = g * _K + b
                gd[b].wait()
                sd.append(pltpu.async_copy(
                    rows_v.at[b], acc_sh.at[dst_v.at[j]], ssems[b], add=True))
            for b in range(_K):
                sd[b].wait()
            return carry

        lax.fori_loop(0, _R // _K, pipe, 0)
        plsc.subcore_barrier()
        pltpu.sync_copy(acc_sh.at[pl.ds(s * _STRIPE, _STRIPE)], strip_v)
        pltpu.sync_copy(strip_v, out_hbm.at[c, pl.ds(s * _STRIPE, _STRIPE)])

    return agg(table, src_rows, dst_rows, zeros)


def _tc_mid_body(xs_ref, p_ref, w2s_ref, b2_ref, h_ref, hs_ref):
    p = p_ref[0] + p_ref[1]
    den = jnp.maximum(p[:, _H:_H + 1], 1.0)
    h = jnp.maximum(xs_ref[...] + p[:, :_H] / den, 0.0)
    h_ref[...] = h
    hs_ref[...] = jnp.dot(h, w2s_ref[...], preferred_element_type=jnp.float32) + b2_ref[...]


def _tc_mid(xs, p32, w2sT, b2r):
    return pl.pallas_call(
        _tc_mid_body,
        grid=(_N // _BLK,),
        in_specs=[
            pl.BlockSpec((_BLK, _H), lambda i: (i, 0)),
            pl.BlockSpec((_NC, _BLK, _W1), lambda i: (0, i, 0)),
            pl.BlockSpec((_H, _C), lambda i: (0, 0)),
            pl.BlockSpec((1, _C), lambda i: (0, 0)),
        ],
        out_specs=[
            pl.BlockSpec((_BLK, _H), lambda i: (i, 0)),
            pl.BlockSpec((_BLK, _C), lambda i: (i, 0)),
        ],
        out_shape=[
            jax.ShapeDtypeStruct((_N, _H), jnp.float32),
            jax.ShapeDtypeStruct((_N, _C), jnp.float32),
        ],
    )(xs, p32, w2sT, b2r)


def _tc_out_body(hs_ref, q_ref, p_ref, w2n_ref, o_ref):
    p = p_ref[0] + p_ref[1]
    den = jnp.maximum(p[:, _H:_H + 1], 1.0)
    agg = q_ref[0] + q_ref[1]
    o_ref[...] = hs_ref[...] + jnp.dot(agg, w2n_ref[...], preferred_element_type=jnp.float32) / den


def _tc_out(hs, q16, p32, w2nT):
    return pl.pallas_call(
        _tc_out_body,
        grid=(_N // _BLK,),
        in_specs=[
            pl.BlockSpec((_BLK, _C), lambda i: (i, 0)),
            pl.BlockSpec((_NC, _BLK, _H), lambda i: (0, i, 0)),
            pl.BlockSpec((_NC, _BLK, _W1), lambda i: (0, i, 0)),
            pl.BlockSpec((_H, _C), lambda i: (0, 0)),
        ],
        out_specs=pl.BlockSpec((_BLK, _C), lambda i: (i, 0)),
        out_shape=jax.ShapeDtypeStruct((_N, _C), jnp.float32),
    )(hs, q16, p32, w2nT)


def kernel(x, edge_index, W1_self, W1_neigh, b1, W2_self, W2_neigh, b2):
    src = edge_index[0].astype(jnp.int32)
    dst = edge_index[1].astype(jnp.int32)
    pad = _EP - _E
    pad_ar = jnp.arange(pad, dtype=jnp.int32)
    src_rows = jnp.concatenate([src, (pad_ar * 127) % _N]).reshape(_NW * _R, _M)
    dst_rows = jnp.concatenate([dst, _N + pad_ar % (_NPAD - _N)]).reshape(_NW * _R, _M)
    wsT = W1_self.T
    wnT = jnp.zeros((_D, _W1), jnp.float32).at[:, :_H].set(W1_neigh.T)
    c1 = jnp.zeros((1, _W1), jnp.float32).at[0, _H].set(1.0)
    z32 = jnp.zeros((_NPAD, _W1), jnp.float32)
    z16 = jnp.zeros((_NPAD, _H), jnp.float32)

    xs, xn = _tc_pre(x, wsT, wnT, b1.reshape(1, _H), c1)
    p32 = _sc_agg(xn, src_rows, dst_rows, z32, _W1)
    h, hs = _tc_mid(xs, p32, W2_self.T, b2.reshape(1, _C))
    q16 = _sc_agg(h, src_rows, dst_rows, z16, _H)
    return _tc_out(hs, q16, p32, W2_neigh.T)
